# baseline (speedup 1.0000x reference)
import jax
import jax.numpy as jnp
from jax.experimental import pallas as pl

NUM_USERS = 80000
NUM_ITEMS = 20000
HOP = 3
ALPHA = 0.5
BATCH = 1024


def _final_body(z_ref, out_ref):
    z = z_ref[...]
    out_ref[...] = jnp.mean(jnp.log1p(jnp.exp(-jnp.abs(z))) + jnp.maximum(z, 0.0), keepdims=True)[:, :1]


def kernel(user_emb, item_emb, A_vals, prior, A_rows, A_cols, users, items, candidates, epoch):
    all_emb = jnp.concatenate([user_emb, item_emb], axis=0)
    embs = [all_emb]
    x = all_emb
    for _ in range(HOP):
        x = jnp.zeros_like(x).at[A_rows].add(A_vals[:, None] * x[A_cols])
        embs.append(x)
    light_out = jnp.mean(jnp.stack(embs, axis=1), axis=1)
    all_users_emb, all_items_emb = light_out[:NUM_USERS], light_out[NUM_USERS:]
    users_emb = all_users_emb[users]
    items_emb = all_items_emb[items]
    sg = users_emb @ all_items_emb.T
    b = jnp.arange(BATCH)
    x_ui = sg[b, items]
    cand_scores = jnp.take_along_axis(sg, candidates, axis=1)
    info = 1.0 - jax.nn.sigmoid(x_ui[:, None] - cand_scores)
    p_fn = prior[candidates]
    F_n = jnp.sum(sg[:, None, :] <= cand_scores[:, :, None], axis=2).astype(jnp.float32) / (NUM_ITEMS + 1)
    unbias = (1.0 - F_n) * (1.0 - p_fn) / (1.0 - F_n - p_fn + 2.0 * F_n * p_fn)
    conditional_risk = (1.0 - unbias) * info - ALPHA * unbias * info
    negatives = candidates[b, jnp.argmin(conditional_risk, axis=1)]
    neg_item_emb = all_items_emb[negatives]
    pos_scores = jnp.sum(users_emb * items_emb, axis=1)
    neg_scores = jnp.sum(users_emb * neg_item_emb, axis=1)
    z = (neg_scores - pos_scores).reshape(BATCH, 1)
    loss = pl.pallas_call(
        _final_body,
        out_shape=jax.ShapeDtypeStruct((1, 1), jnp.float32),
    )(z)
    return loss.reshape(())


# speedup vs baseline: 4.0757x; 4.0757x over previous
import functools

import jax
import jax.numpy as jnp
from jax import lax
from jax.experimental import pallas as pl
from jax.experimental.pallas import tpu as pltpu
from jax.experimental.pallas import tpu_sc as plsc

NUM_USERS = 80000
NUM_ITEMS = 20000
DIM = 64
HOP = 3
NUM_NEG = 8
ALPHA = 0.5
N_NODES = NUM_USERS + NUM_ITEMS
NPAD = 100096
N_EDGES = 1000000
BATCH = 1024

G = 4
GD = 16
EPAD = 1 << 20
NS = 16
NC = 2
E_TILE = EPAD // NS
NB = 512
NBLK = E_TILE // NB
ROWS_TILE = NPAD // NS
ZROWS = 782

_mesh = plsc.VectorSubcoreMesh(core_axis_name="c", subcore_axis_name="s")


def _hop_body(x_hbm, rows_hbm, cols_hbm, vals_hbm, out_hbm,
              y_sh, colsv, rowsv, valsv, gbuf, zbuf, sem):
    c = lax.axis_index("c")
    s = lax.axis_index("s")
    zero16 = jnp.zeros((GD,), jnp.float32)

    @plsc.parallel_loop(0, ZROWS, unroll=8)
    def _(i):
        zbuf[i, :] = zero16

    base = s * ROWS_TILE
    for g_local in range(2):
        g = c * 2 + g_local
        gN = g * NPAD
        for z in range(ROWS_TILE // ZROWS):
            pltpu.sync_copy(zbuf, y_sh.at[pl.ds(base + z * ZROWS, ZROWS)])
        plsc.subcore_barrier()

        def blk_body(t, _):
            off = s * E_TILE + t * NB
            blkrow = s * (E_TILE // 128) + t * (NB // 128)
            pltpu.sync_copy(cols_hbm.at[pl.ds(blkrow, NB // 128)], colsv)
            pltpu.sync_copy(rows_hbm.at[pl.ds(blkrow, NB // 128)], rowsv)
            pltpu.sync_copy(vals_hbm.at[pl.ds(off, NB)], valsv)
            for j in range(NB // 128):
                for q in range(8):
                    colsv[j, pl.ds(q * 16, 16)] = colsv[j, pl.ds(q * 16, 16)] + gN
            cps = [pltpu.async_copy(x_hbm.at[colsv.at[j]],
                                    gbuf.at[pl.ds(j * 128, 128)], sem)
                   for j in range(NB // 128)]
            for cp in cps:
                cp.wait()

            @plsc.parallel_loop(0, NB // 16, unroll=2)
            def _(i):
                e0 = i * 16
                vv = valsv[pl.ds(e0, 16)]
                for l in range(16):
                    gbuf[e0 + l, :] = gbuf[e0 + l, :] * vv[l]

            for j in range(NB // 128):
                pltpu.sync_copy(gbuf.at[pl.ds(j * 128, 128)],
                                y_sh.at[rowsv.at[j]], add=True)
            return ()

        lax.fori_loop(0, NBLK, blk_body, (), unroll=False)
        plsc.subcore_barrier()
        pltpu.sync_copy(y_sh.at[pl.ds(base, ROWS_TILE)],
                        out_hbm.at[pl.ds(gN + base, ROWS_TILE)])
        plsc.subcore_barrier()


_hop = pl.kernel(
    _hop_body,
    out_type=jax.ShapeDtypeStruct((G * NPAD, GD), jnp.float32),
    mesh=_mesh,
    compiler_params=pltpu.CompilerParams(use_tc_tiling_on_sc=False),
    scratch_types=[
        pltpu.VMEM_SHARED((NPAD, GD), jnp.float32),
        pltpu.VMEM((NB // 128, 128), jnp.int32),
        pltpu.VMEM((NB // 128, 128), jnp.int32),
        pltpu.VMEM((NB,), jnp.float32),
        pltpu.VMEM((NB, GD), jnp.float32),
        pltpu.VMEM((ZROWS, GD), jnp.float32),
        pltpu.SemaphoreType.DMA,
    ],
)


def _gather_body(light_hbm, prior_hbm, users_hbm, items_hbm, cand_hbm,
                 u_out, i_out, c_out, p_out,
                 idxv, rbuf, pv, pidx, pbuf, sem):
    c = lax.axis_index("c")
    s = lax.axis_index("s")
    w = s * NC + c
    g = w // 8
    part = w % 8
    gN = g * NPAD

    def gather_rows(src2d_hbm, src_row, shift, dst, dst_off):
        pltpu.sync_copy(src2d_hbm.at[pl.ds(src_row, 1)], idxv)
        for q in range(8):
            idxv[0, pl.ds(q * 16, 16)] = idxv[0, pl.ds(q * 16, 16)] + shift
        pltpu.async_copy(light_hbm.at[idxv.at[0]], rbuf, sem).wait()
        pltpu.sync_copy(rbuf, dst.at[pl.ds(dst_off, 128)])

    gather_rows(users_hbm, part, gN, u_out, g * BATCH + part * 128)
    gather_rows(items_hbm, part, gN + NUM_USERS, i_out, g * BATCH + part * 128)
    for r in range(8):
        row = part * 8 + r
        gather_rows(cand_hbm, row, gN + NUM_USERS, c_out,
                    g * (BATCH * NUM_NEG) + row * 128)
    pltpu.sync_copy(prior_hbm, pv)
    pltpu.sync_copy(cand_hbm.at[pl.ds(w * 2, 2)], pidx)
    for j in range(2):
        for q in range(8):
            ids = pidx[j, pl.ds(q * 16, 16)]
            pbuf[j, pl.ds(q * 16, 16)] = plsc.load_gather(pv, [ids])
    pltpu.sync_copy(pbuf, p_out.at[pl.ds(w * 2, 2)])


_gather = pl.kernel(
    _gather_body,
    out_type=(
        jax.ShapeDtypeStruct((G * BATCH, GD), jnp.float32),
        jax.ShapeDtypeStruct((G * BATCH, GD), jnp.float32),
        jax.ShapeDtypeStruct((G * BATCH * NUM_NEG, GD), jnp.float32),
        jax.ShapeDtypeStruct((BATCH * NUM_NEG // 128, 128), jnp.float32),
    ),
    mesh=_mesh,
    compiler_params=pltpu.CompilerParams(use_tc_tiling_on_sc=False,
                                          needs_layout_passes=False),
    scratch_types=[
        pltpu.VMEM((1, 128), jnp.int32),
        pltpu.VMEM((128, GD), jnp.float32),
        pltpu.VMEM((NUM_ITEMS,), jnp.float32),
        pltpu.VMEM((2, 128), jnp.int32),
        pltpu.VMEM((2, 128), jnp.float32),
        pltpu.SemaphoreType.DMA,
    ],
)


def _mean_body(a, b, c, d, o):
    o[...] = (a[...] + b[...] + c[...] + d[...]) * 0.25


def _mean(x0, x1, x2, x3):
    r = G * NPAD * GD // 256
    blk = r // 8
    spec = pl.BlockSpec((blk, 256), lambda i: (i, 0))
    return pl.pallas_call(
        _mean_body,
        grid=(8,),
        in_specs=[spec] * 4,
        out_specs=spec,
        out_shape=jax.ShapeDtypeStruct((r, 256), jnp.float32),
    )(x0.reshape(r, 256), x1.reshape(r, 256), x2.reshape(r, 256),
      x3.reshape(r, 256))


def _candscore_body(u, i, cnd, xui, ct):
    uu = u[...]
    xui[...] = jnp.sum(uu * i[...], axis=(0, 2))[None, :]
    ct[...] = jnp.sum(uu[:, None, :, :] * cnd[...], axis=(0, 3))


def _candscore(u4, i4, c4):
    return pl.pallas_call(
        _candscore_body,
        out_shape=(jax.ShapeDtypeStruct((1, BATCH), jnp.float32),
                   jax.ShapeDtypeStruct((NUM_NEG, BATCH), jnp.float32)),
    )(u4, i4, c4)


def _count_body(li, u, ct, o):
    i = pl.program_id(0)
    s = jnp.zeros((BATCH, 2000), jnp.float32)
    for g in range(G):
        s = s + lax.dot_general(u[g], li[g], (((1,), (1,)), ((), ())),
                                preferred_element_type=jnp.float32)
    cnt = jnp.concatenate(
        [jnp.sum((s <= ct[k, :][:, None]).astype(jnp.float32), axis=1)[None, :]
         for k in range(NUM_NEG)], axis=0)

    @pl.when(i == 0)
    def _():
        o[...] = cnt

    @pl.when(i != 0)
    def _():
        o[...] = o[...] + cnt


def _count(li, u4, ct):
    return pl.pallas_call(
        _count_body,
        grid=(10,),
        in_specs=[pl.BlockSpec((G, 2000, GD), lambda i: (0, i, 0)),
                  pl.BlockSpec((G, BATCH, GD), lambda i: (0, 0, 0)),
                  pl.BlockSpec((NUM_NEG, BATCH), lambda i: (0, 0))],
        out_specs=pl.BlockSpec((NUM_NEG, BATCH), lambda i: (0, 0)),
        out_shape=jax.ShapeDtypeStruct((NUM_NEG, BATCH), jnp.float32),
    )(li, u4, ct)


def _final_body(cnt, ct, xui, pfn, o):
    f = cnt[...] / (NUM_ITEMS + 1)
    cs = ct[...]
    p = pfn[...]
    info = 1.0 - jax.nn.sigmoid(xui[...] - cs)
    unbias = (1.0 - f) * (1.0 - p) / (1.0 - f - p + 2.0 * f * p)
    risk = info * (1.0 - (1.0 + ALPHA) * unbias)
    best = risk[0, :]
    bestsc = cs[0, :]
    for k in range(1, NUM_NEG):
        lt = risk[k, :] < best
        bestsc = jnp.where(lt, cs[k, :], bestsc)
        best = jnp.where(lt, risk[k, :], best)
    z = bestsc - xui[0, :]
    o[...] = jnp.mean(jnp.log1p(jnp.exp(-jnp.abs(z))) + jnp.maximum(z, 0.0)).reshape(1, 1)


def _final(cnt, ct, xui, pfn):
    return pl.pallas_call(
        _final_body,
        out_shape=jax.ShapeDtypeStruct((1, 1), jnp.float32),
    )(cnt, ct, xui, pfn)


def kernel(user_emb, item_emb, A_vals, prior, A_rows, A_cols, users, items,
           candidates, epoch):
    all_emb = jnp.concatenate([
        user_emb, item_emb,
        jnp.zeros((NPAD - N_NODES, DIM), jnp.float32)], axis=0)
    x0 = all_emb.reshape(NPAD, G, GD).transpose(1, 0, 2).reshape(G * NPAD, GD)

    npad = EPAD - N_EDGES
    padr = (jnp.arange(npad, dtype=jnp.int32) * 17) % N_NODES
    rows32 = jnp.concatenate([A_rows.astype(jnp.int32), padr]).reshape(EPAD // 128, 128)
    cols32 = jnp.concatenate([A_cols.astype(jnp.int32), padr]).reshape(EPAD // 128, 128)
    vals = jnp.concatenate([A_vals, jnp.zeros((npad,), jnp.float32)])

    x1 = _hop(x0, rows32, cols32, vals)
    x2 = _hop(x1, rows32, cols32, vals)
    x3 = _hop(x2, rows32, cols32, vals)
    light = _mean(x0, x1, x2, x3).reshape(G * NPAD, GD)

    users2d = users.astype(jnp.int32).reshape(8, 128)
    items2d = items.astype(jnp.int32).reshape(8, 128)
    cand2d = candidates.astype(jnp.int32).T.reshape(NUM_NEG * BATCH // 128, 128)

    u_f, i_f, c_f, p_f = _gather(light, prior, users2d, items2d, cand2d)
    u4 = u_f.reshape(G, BATCH, GD)
    i4 = i_f.reshape(G, BATCH, GD)
    c4 = c_f.reshape(G, NUM_NEG, BATCH, GD)
    pfn = p_f.reshape(NUM_NEG, BATCH)

    xui, ct = _candscore(u4, i4, c4)
    li = light.reshape(G, NPAD, GD)[:, NUM_USERS:N_NODES, :]
    cnt = _count(li, u4, ct)
    loss = _final(cnt, ct, xui, pfn)
    return loss.reshape(())


# speedup vs baseline: 8.5684x; 2.1023x over previous
import functools

import jax
import jax.numpy as jnp
from jax import lax
from jax.experimental import pallas as pl
from jax.experimental.pallas import tpu as pltpu
from jax.experimental.pallas import tpu_sc as plsc

NUM_USERS = 80000
NUM_ITEMS = 20000
DIM = 64
HOP = 3
NUM_NEG = 8
ALPHA = 0.5
N_NODES = NUM_USERS + NUM_ITEMS
NPAD = 100096
N_EDGES = 1000000
BATCH = 1024

G = 4
GD = 16
EPAD = 1 << 20
NS = 16
NC = 2
E_TILE = EPAD // NS
NB = 512
NBLK = E_TILE // NB
ROWS_TILE = NPAD // NS
ZROWS = 782

_mesh = plsc.VectorSubcoreMesh(core_axis_name="c", subcore_axis_name="s")


def _hop_body(x_hbm, rows_hbm, cols_hbm, vals_hbm, out_hbm,
              y_sh, colsv, rowsv, valsv, gbuf, semi, semg):
    c = lax.axis_index("c")
    s = lax.axis_index("s")
    zero16 = jnp.zeros((GD,), jnp.float32)
    base = s * ROWS_TILE
    NJ = NB // 128

    def idx_issue(blk, m):
        blkrow = s * (E_TILE // 128) + blk * NJ
        off = s * E_TILE + blk * NB
        pltpu.async_copy(cols_hbm.at[pl.ds(blkrow, NJ)],
                         colsv.at[pl.ds(m * NJ, NJ)], semi)
        pltpu.async_copy(rows_hbm.at[pl.ds(blkrow, NJ)],
                         rowsv.at[pl.ds(m * NJ, NJ)], semi)
        pltpu.async_copy(vals_hbm.at[pl.ds(off, NB)],
                         valsv.at[pl.ds(m * NB, NB)], semi)

    def idx_drain(m):
        pltpu.make_async_copy(cols_hbm.at[pl.ds(0, NJ)],
                              colsv.at[pl.ds(m * NJ, NJ)], semi).wait()
        pltpu.make_async_copy(rows_hbm.at[pl.ds(0, NJ)],
                              rowsv.at[pl.ds(m * NJ, NJ)], semi).wait()
        pltpu.make_async_copy(vals_hbm.at[pl.ds(0, NB)],
                              valsv.at[pl.ds(m * NB, NB)], semi).wait()

    def adjust_cols(m, gN):
        for j in range(NJ):
            for q in range(8):
                colsv[m * NJ + j, pl.ds(q * 16, 16)] = (
                    colsv[m * NJ + j, pl.ds(q * 16, 16)] + gN)

    def gather_issue(m, p):
        for j in range(NJ):
            pltpu.async_copy(x_hbm.at[colsv.at[m * NJ + j]],
                             gbuf.at[pl.ds(p * NB + j * 128, 128)], semg)

    def gather_drain(p):
        for j in range(NJ):
            pltpu.make_async_copy(x_hbm.at[pl.ds(0, 128)],
                                  gbuf.at[pl.ds(p * NB + j * 128, 128)],
                                  semg).wait()

    for g_local in range(2):
        g = c * 2 + g_local
        gN = g * NPAD
        @plsc.parallel_loop(0, 2 * NB, unroll=8)
        def _(i):
            gbuf[i, :] = zero16

        for z in range(ROWS_TILE // (2 * NB)):
            pltpu.sync_copy(gbuf, y_sh.at[pl.ds(base + z * 2 * NB, 2 * NB)])
        rem = ROWS_TILE % (2 * NB)
        if rem:
            pltpu.sync_copy(gbuf.at[pl.ds(0, rem)],
                            y_sh.at[pl.ds(base + ROWS_TILE - rem, rem)])
        plsc.subcore_barrier()

        idx_issue(0, 0)
        idx_drain(0)
        adjust_cols(0, gN)
        gather_issue(0, 0)
        idx_issue(1, 1)

        def blk_body(t, _):
            m = t % 3
            mp = (t - 1) % 3
            p = t % 2
            q = 1 - p
            blk = t % NBLK
            idx_drain(m)
            adjust_cols(m, gN)
            gather_drain(q)
            gather_issue(m, p)
            idx_issue((t + 1) % NBLK, (t + 1) % 3)

            @plsc.parallel_loop(0, NB // 16, unroll=2)
            def _(i):
                e0 = i * 16
                vv = valsv[pl.ds(mp * NB + e0, 16)]
                for l in range(16):
                    gbuf[q * NB + e0 + l, :] = gbuf[q * NB + e0 + l, :] * vv[l]

            for j in range(NJ):
                pltpu.sync_copy(gbuf.at[pl.ds(q * NB + j * 128, 128)],
                                y_sh.at[rowsv.at[mp * NJ + j]], add=True)
            return ()

        lax.fori_loop(1, NBLK + 1, blk_body, (), unroll=False)
        gather_drain(0)
        idx_drain((NBLK + 1) % 3)
        plsc.subcore_barrier()
        pltpu.sync_copy(y_sh.at[pl.ds(base, ROWS_TILE)],
                        out_hbm.at[pl.ds(gN + base, ROWS_TILE)])
        plsc.subcore_barrier()


_hop = pl.kernel(
    _hop_body,
    out_type=jax.ShapeDtypeStruct((G * NPAD, GD), jnp.float32),
    mesh=_mesh,
    compiler_params=pltpu.CompilerParams(use_tc_tiling_on_sc=False),
    scratch_types=[
        pltpu.VMEM_SHARED((NPAD, GD), jnp.float32),
        pltpu.VMEM((3 * (NB // 128), 128), jnp.int32),
        pltpu.VMEM((3 * (NB // 128), 128), jnp.int32),
        pltpu.VMEM((3 * NB,), jnp.float32),
        pltpu.VMEM((2 * NB, GD), jnp.float32),
        pltpu.SemaphoreType.DMA,
        pltpu.SemaphoreType.DMA,
    ],
)


def _gather_body(light_hbm, prior_hbm, users_hbm, items_hbm, cand_hbm,
                 u_out, i_out, c_out, p_out,
                 idxv, rbuf, pv, pidx, pbuf, sem):
    c = lax.axis_index("c")
    s = lax.axis_index("s")
    w = s * NC + c
    g = w // 8
    part = w % 8
    gN = g * NPAD

    def gather_rows(src2d_hbm, src_row, shift, dst, dst_off):
        pltpu.sync_copy(src2d_hbm.at[pl.ds(src_row, 1)], idxv)
        for q in range(8):
            idxv[0, pl.ds(q * 16, 16)] = idxv[0, pl.ds(q * 16, 16)] + shift
        pltpu.async_copy(light_hbm.at[idxv.at[0]], rbuf, sem).wait()
        pltpu.sync_copy(rbuf, dst.at[pl.ds(dst_off, 128)])

    gather_rows(users_hbm, part, gN, u_out, g * BATCH + part * 128)
    gather_rows(items_hbm, part, gN + NUM_USERS, i_out, g * BATCH + part * 128)
    for r in range(8):
        row = part * 8 + r
        gather_rows(cand_hbm, row, gN + NUM_USERS, c_out,
                    g * (BATCH * NUM_NEG) + row * 128)
    pltpu.sync_copy(prior_hbm, pv)
    pltpu.sync_copy(cand_hbm.at[pl.ds(w * 2, 2)], pidx)
    for j in range(2):
        for q in range(8):
            ids = pidx[j, pl.ds(q * 16, 16)]
            pbuf[j, pl.ds(q * 16, 16)] = plsc.load_gather(pv, [ids])
    pltpu.sync_copy(pbuf, p_out.at[pl.ds(w * 2, 2)])


_gather = pl.kernel(
    _gather_body,
    out_type=(
        jax.ShapeDtypeStruct((G * BATCH, GD), jnp.float32),
        jax.ShapeDtypeStruct((G * BATCH, GD), jnp.float32),
        jax.ShapeDtypeStruct((G * BATCH * NUM_NEG, GD), jnp.float32),
        jax.ShapeDtypeStruct((BATCH * NUM_NEG // 128, 128), jnp.float32),
    ),
    mesh=_mesh,
    compiler_params=pltpu.CompilerParams(use_tc_tiling_on_sc=False,
                                          needs_layout_passes=False),
    scratch_types=[
        pltpu.VMEM((1, 128), jnp.int32),
        pltpu.VMEM((128, GD), jnp.float32),
        pltpu.VMEM((NUM_ITEMS,), jnp.float32),
        pltpu.VMEM((2, 128), jnp.int32),
        pltpu.VMEM((2, 128), jnp.float32),
        pltpu.SemaphoreType.DMA,
    ],
)


def _mean_body(a, b, c, d, o):
    o[...] = (a[...] + b[...] + c[...] + d[...]) * 0.25


def _mean(x0, x1, x2, x3):
    r = G * NPAD * GD // 256
    blk = r // 8
    spec = pl.BlockSpec((blk, 256), lambda i: (i, 0))
    return pl.pallas_call(
        _mean_body,
        grid=(8,),
        in_specs=[spec] * 4,
        out_specs=spec,
        out_shape=jax.ShapeDtypeStruct((r, 256), jnp.float32),
    )(x0.reshape(r, 256), x1.reshape(r, 256), x2.reshape(r, 256),
      x3.reshape(r, 256))


def _candscore_body(u, i, cnd, xui, ct):
    uu = u[...]
    xui[...] = jnp.sum(uu * i[...], axis=(0, 2))[None, :]
    ct[...] = jnp.sum(uu[:, None, :, :] * cnd[...], axis=(0, 3))


def _candscore(u4, i4, c4):
    return pl.pallas_call(
        _candscore_body,
        out_shape=(jax.ShapeDtypeStruct((1, BATCH), jnp.float32),
                   jax.ShapeDtypeStruct((NUM_NEG, BATCH), jnp.float32)),
    )(u4, i4, c4)


def _count_body(li, u, ct, o):
    i = pl.program_id(0)
    s = jnp.zeros((BATCH, 2000), jnp.float32)
    for g in range(G):
        s = s + lax.dot_general(u[g], li[g], (((1,), (1,)), ((), ())),
                                preferred_element_type=jnp.float32)
    cnt = jnp.concatenate(
        [jnp.sum((s <= ct[k, :][:, None]).astype(jnp.float32), axis=1)[None, :]
         for k in range(NUM_NEG)], axis=0)

    @pl.when(i == 0)
    def _():
        o[...] = cnt

    @pl.when(i != 0)
    def _():
        o[...] = o[...] + cnt


def _count(li, u4, ct):
    return pl.pallas_call(
        _count_body,
        grid=(10,),
        in_specs=[pl.BlockSpec((G, 2000, GD), lambda i: (0, i, 0)),
                  pl.BlockSpec((G, BATCH, GD), lambda i: (0, 0, 0)),
                  pl.BlockSpec((NUM_NEG, BATCH), lambda i: (0, 0))],
        out_specs=pl.BlockSpec((NUM_NEG, BATCH), lambda i: (0, 0)),
        out_shape=jax.ShapeDtypeStruct((NUM_NEG, BATCH), jnp.float32),
    )(li, u4, ct)


def _final_body(cnt, ct, xui, pfn, o):
    f = cnt[...] / (NUM_ITEMS + 1)
    cs = ct[...]
    p = pfn[...]
    info = 1.0 - jax.nn.sigmoid(xui[...] - cs)
    unbias = (1.0 - f) * (1.0 - p) / (1.0 - f - p + 2.0 * f * p)
    risk = info * (1.0 - (1.0 + ALPHA) * unbias)
    best = risk[0, :]
    bestsc = cs[0, :]
    for k in range(1, NUM_NEG):
        lt = risk[k, :] < best
        bestsc = jnp.where(lt, cs[k, :], bestsc)
        best = jnp.where(lt, risk[k, :], best)
    z = bestsc - xui[0, :]
    o[...] = jnp.mean(jnp.log1p(jnp.exp(-jnp.abs(z))) + jnp.maximum(z, 0.0)).reshape(1, 1)


def _final(cnt, ct, xui, pfn):
    return pl.pallas_call(
        _final_body,
        out_shape=jax.ShapeDtypeStruct((1, 1), jnp.float32),
    )(cnt, ct, xui, pfn)


def kernel(user_emb, item_emb, A_vals, prior, A_rows, A_cols, users, items,
           candidates, epoch):
    all_emb = jnp.concatenate([
        user_emb, item_emb,
        jnp.zeros((NPAD - N_NODES, DIM), jnp.float32)], axis=0)
    x0 = all_emb.reshape(NPAD, G, GD).transpose(1, 0, 2).reshape(G * NPAD, GD)

    npad = EPAD - N_EDGES
    padr = (jnp.arange(npad, dtype=jnp.int32) * 17) % N_NODES
    rows32 = jnp.concatenate([A_rows.astype(jnp.int32), padr]).reshape(EPAD // 128, 128)
    cols32 = jnp.concatenate([A_cols.astype(jnp.int32), padr]).reshape(EPAD // 128, 128)
    vals = jnp.concatenate([A_vals, jnp.zeros((npad,), jnp.float32)])

    x1 = _hop(x0, rows32, cols32, vals)
    x2 = _hop(x1, rows32, cols32, vals)
    x3 = _hop(x2, rows32, cols32, vals)
    light = _mean(x0, x1, x2, x3).reshape(G * NPAD, GD)

    users2d = users.astype(jnp.int32).reshape(8, 128)
    items2d = items.astype(jnp.int32).reshape(8, 128)
    cand2d = candidates.astype(jnp.int32).T.reshape(NUM_NEG * BATCH // 128, 128)

    u_f, i_f, c_f, p_f = _gather(light, prior, users2d, items2d, cand2d)
    u4 = u_f.reshape(G, BATCH, GD)
    i4 = i_f.reshape(G, BATCH, GD)
    c4 = c_f.reshape(G, NUM_NEG, BATCH, GD)
    pfn = p_f.reshape(NUM_NEG, BATCH)

    xui, ct = _candscore(u4, i4, c4)
    li = light.reshape(G, NPAD, GD)[:, NUM_USERS:N_NODES, :]
    cnt = _count(li, u4, ct)
    loss = _final(cnt, ct, xui, pfn)
    return loss.reshape(())
